# baseline (speedup 1.0000x reference)
import functools

import jax
import jax.numpy as jnp
from jax import lax
from jax.experimental import pallas as pl
from jax.experimental.pallas import tpu as pltpu
from jax.experimental.pallas import tpu_sc as plsc

N = 10000
E = 320000
F_IN = 128
H = 256
NG = 64
NC = 2
NS = 16
HH = H // 2

CW = 32
ROWS = E // CW
RPT = ROWS // NS
GRP = 5
NGRP = RPT // GRP
NPAIR = NGRP // 2

CWD = 40
ROWS_D = E // CWD
RPT_D = ROWS_D // (NC * NS)
NGRP_D = RPT_D // GRP
NPT = 624
TAIL = N - NS * NPT

BN_R = 1000
NBLK = N // BN_R

_MESH = dict(core_axis_name="c", subcore_axis_name="s",
             num_cores=NC, num_subcores=NS)


def _part_copy(src, dst, s, src_base, dst_base):
    pltpu.sync_copy(src.at[pl.ds(src_base + s * NPT, NPT)],
                    dst.at[pl.ds(dst_base + s * NPT, NPT)])

    @pl.when(s == NS - 1)
    def _():
        pltpu.sync_copy(src.at[pl.ds(src_base + NS * NPT, TAIL)],
                        dst.at[pl.ds(dst_base + NS * NPT, TAIL)])


def _deg_body(dstR, z16, ones80, out, acc, ones_v, didx, sem_i, sem_s):
    c = lax.axis_index("c")
    s = lax.axis_index("s")
    w = c * NS + s
    _part_copy(z16, acc, s, 0, 0)
    pltpu.sync_copy(ones80, ones_v)
    plsc.subcore_barrier()

    def grp(g, carry):
        base = w * RPT_D + g * GRP
        pltpu.async_copy(dstR.at[pl.ds(base, GRP)], didx, sem_i).wait()
        ss = [pltpu.async_copy(ones_v, acc.at[didx.at[b, 0]], sem_s, add=True)
              for b in range(GRP)]
        for d in ss:
            d.wait()
        return carry

    lax.fori_loop(0, NGRP_D, grp, 0)
    plsc.subcore_barrier()
    _part_copy(acc, out, s, 0, c * N)


def _deg_call(dstR, z16, ones80):
    return pl.kernel(
        _deg_body,
        out_type=jax.ShapeDtypeStruct((NC * N, 128), jnp.float32),
        mesh=plsc.VectorSubcoreMesh(**_MESH),
        scratch_types=[
            pltpu.VMEM_SHARED((N, 128), jnp.float32),
            pltpu.VMEM((CWD, 128), jnp.float32),
            pltpu.VMEM((GRP, 1, CWD), jnp.int32),
            pltpu.SemaphoreType.DMA,
            pltpu.SemaphoreType.DMA,
        ],
    )(dstR, z16, ones80)


def _scat_body(xs, srcR2, dstR, out, acc,
               sidxA, didxA, bufsA, sidxB, didxB, bufsB,
               sem_i, sem_g, sem_sa, sem_sb):
    c = lax.axis_index("c")
    s = lax.axis_index("s")
    row0 = s * RPT

    def idx_load(sidx, didx, base):
        d1 = pltpu.async_copy(srcR2.at[c, pl.ds(base, GRP)], sidx, sem_i)
        d2 = pltpu.async_copy(dstR.at[pl.ds(base, GRP)], didx, sem_i)
        d1.wait()
        d2.wait()

    def gathers(sidx, bufs):
        for b in range(GRP):
            pltpu.async_copy(xs.at[sidx.at[b, 0]], bufs[b], sem_g)

    def wait_gathers(sidx, bufs):
        for b in range(GRP):
            pltpu.make_async_copy(xs.at[sidx.at[b, 0]], bufs[b], sem_g).wait()

    def scatters(didx, bufs, sem):
        return [pltpu.async_copy(bufs[b], acc.at[didx.at[b, 0]], sem,
                                 add=True)
                for b in range(GRP)]

    idx_load(sidxA, didxA, row0)
    gathers(sidxA, bufsA)
    idx_load(sidxB, didxB, row0 + GRP)
    _part_copy(xs, acc, s, c * N, 0)
    plsc.subcore_barrier()

    def pair(i, carry):
        base_a2 = row0 + (2 * i + 2) * GRP
        base_a2 = lax.min(base_a2, ROWS - GRP)
        base_b2 = lax.min(base_a2 + GRP, ROWS - GRP)
        wait_gathers(sidxA, bufsA)
        sA = scatters(didxA, bufsA, sem_sa)
        gathers(sidxB, bufsB)
        wait_gathers(sidxB, bufsB)
        sB = scatters(didxB, bufsB, sem_sb)
        for d in sA:
            d.wait()
        idx_load(sidxA, didxA, base_a2)
        gathers(sidxA, bufsA)
        for d in sB:
            d.wait()
        idx_load(sidxB, didxB, base_b2)
        return carry

    lax.fori_loop(0, NPAIR, pair, 0)
    wait_gathers(sidxA, bufsA)
    for d in scatters(didxA, bufsA, sem_sa):
        d.wait()
    plsc.subcore_barrier()
    _part_copy(acc, out, s, 0, c * N)


def _scat_call(xs_flat, srcR2, dstR):
    return pl.kernel(
        _scat_body,
        out_type=jax.ShapeDtypeStruct((NC * N, HH), jnp.float32),
        mesh=plsc.VectorSubcoreMesh(**_MESH),
        scratch_types=[
            pltpu.VMEM_SHARED((N, HH), jnp.float32),
            pltpu.VMEM((GRP, 1, CW), jnp.int32),
            pltpu.VMEM((GRP, 1, CW), jnp.int32),
            [pltpu.VMEM((CW, HH), jnp.float32) for _ in range(GRP)],
            pltpu.VMEM((GRP, 1, CW), jnp.int32),
            pltpu.VMEM((GRP, 1, CW), jnp.int32),
            [pltpu.VMEM((CW, HH), jnp.float32) for _ in range(GRP)],
            pltpu.SemaphoreType.DMA,
            pltpu.SemaphoreType.DMA,
            pltpu.SemaphoreType.DMA,
            pltpu.SemaphoreType.DMA,
        ],
    )(xs_flat, srcR2, dstR)


def _tc1_body(x_ref, dega_ref, degb_ref, w1_ref, xs_ref, dinv_ref):
    deg = dega_ref[:, :1] + degb_ref[:, :1] + 1.0
    dinv = lax.rsqrt(deg)
    hw = jnp.dot(x_ref[...], w1_ref[...], preferred_element_type=jnp.float32)
    xs = hw * dinv
    xs_ref[0] = xs[:, :HH]
    xs_ref[1] = xs[:, HH:]
    dinv_ref[...] = jnp.broadcast_to(dinv, (BN_R, HH))


def _tc1_call(x, deg2, W1):
    return pl.pallas_call(
        _tc1_body,
        grid=(NBLK,),
        in_specs=[
            pl.BlockSpec((BN_R, F_IN), lambda i: (i, 0)),
            pl.BlockSpec((BN_R, 128), lambda i: (i, 0)),
            pl.BlockSpec((BN_R, 128), lambda i: (i + NBLK, 0)),
            pl.BlockSpec((F_IN, H), lambda i: (0, 0)),
        ],
        out_specs=[
            pl.BlockSpec((NC, BN_R, HH), lambda i: (0, i, 0)),
            pl.BlockSpec((BN_R, HH), lambda i: (i, 0)),
        ],
        out_shape=[
            jax.ShapeDtypeStruct((NC, N, HH), jnp.float32),
            jax.ShapeDtypeStruct((N, HH), jnp.float32),
        ],
    )(x, deg2, deg2, W1)


def _tcmid_body(acc_ref, dinv_ref, a_ref, b_ref, w_ref, xs_ref):
    dinv = dinv_ref[:, :1]
    y0 = acc_ref[0] * dinv * a_ref[:, :HH] + b_ref[:, :HH]
    y1 = acc_ref[1] * dinv * a_ref[:, HH:] + b_ref[:, HH:]
    h = jax.nn.relu(jnp.concatenate([y0, y1], axis=1))
    hw = jnp.dot(h, w_ref[...], preferred_element_type=jnp.float32)
    xs = hw * dinv
    xs_ref[0] = xs[:, :HH]
    xs_ref[1] = xs[:, HH:]


def _tcmid_call(acc3d, dinv128, A, B, W):
    return pl.pallas_call(
        _tcmid_body,
        grid=(NBLK,),
        in_specs=[
            pl.BlockSpec((NC, BN_R, HH), lambda i: (0, i, 0)),
            pl.BlockSpec((BN_R, HH), lambda i: (i, 0)),
            pl.BlockSpec((1, H), lambda i: (0, 0)),
            pl.BlockSpec((1, H), lambda i: (0, 0)),
            pl.BlockSpec((H, H), lambda i: (0, 0)),
        ],
        out_specs=pl.BlockSpec((NC, BN_R, HH), lambda i: (0, i, 0)),
        out_shape=jax.ShapeDtypeStruct((NC, N, HH), jnp.float32),
    )(acc3d, dinv128, A, B, W)


def _pool_body(acc_ref, dinv_ref, b3_ref, batch_ref, out_ref, sums, cnts):
    i = pl.program_id(0)
    dinv = dinv_ref[:, :1]
    o0 = acc_ref[0] * dinv + b3_ref[:, :HH]
    o1 = acc_ref[1] * dinv + b3_ref[:, HH:]
    out3 = jnp.concatenate([o0, o1], axis=1)
    ids = batch_ref[0, 0, :]
    gid = lax.broadcasted_iota(jnp.int32, (NG, BN_R), 0)
    oh = (gid == jnp.broadcast_to(ids[None, :], (NG, BN_R))
          ).astype(jnp.float32)

    @pl.when(i == 0)
    def _():
        sums[...] = jnp.zeros_like(sums)
        cnts[...] = jnp.zeros_like(cnts)

    sums[...] += jnp.dot(oh, out3, preferred_element_type=jnp.float32)
    cnts[...] += jnp.broadcast_to(
        jnp.sum(oh, axis=1, keepdims=True), (NG, HH))

    @pl.when(i == NBLK - 1)
    def _():
        out_ref[...] = sums[...] / jnp.maximum(cnts[:, :1], 1.0)


def _pool_call(acc3d, dinv128, b3r, batch3):
    return pl.pallas_call(
        _pool_body,
        grid=(NBLK,),
        in_specs=[
            pl.BlockSpec((NC, BN_R, HH), lambda i: (0, i, 0)),
            pl.BlockSpec((BN_R, HH), lambda i: (i, 0)),
            pl.BlockSpec((1, H), lambda i: (0, 0)),
            pl.BlockSpec((1, 1, BN_R), lambda i: (i, 0, 0)),
        ],
        out_specs=pl.BlockSpec((NG, H), lambda i: (0, 0)),
        out_shape=jax.ShapeDtypeStruct((NG, H), jnp.float32),
        scratch_shapes=[
            pltpu.VMEM((NG, H), jnp.float32),
            pltpu.VMEM((NG, HH), jnp.float32),
        ],
    )(acc3d, dinv128, b3r, batch3)


@jax.jit
def kernel(x, edge_index, batch, W1, b1, W2, b2, W3, b3,
           g1, be1, rm1, rv1, g2, be2, rm2, rv2):
    src = edge_index[0].reshape(ROWS, 1, CW)
    dstR = edge_index[1].reshape(ROWS, 1, CW)
    dstRd = edge_index[1].reshape(ROWS_D, 1, CWD)
    srcR2 = jnp.stack([src, src + N])
    batch3 = batch.reshape(NBLK, 1, BN_R)
    z16 = jnp.zeros((N, 128), jnp.float32)
    ones80 = jnp.ones((CWD, 128), jnp.float32)

    s1 = g1 * lax.rsqrt(rv1 + 1e-5)
    A1 = s1.reshape(1, H)
    B1 = ((b1 - rm1) * s1 + be1).reshape(1, H)
    s2 = g2 * lax.rsqrt(rv2 + 1e-5)
    A2 = s2.reshape(1, H)
    B2 = ((b2 - rm2) * s2 + be2).reshape(1, H)
    b3r = b3.reshape(1, H)

    deg2 = _deg_call(dstRd, z16, ones80)
    xs1, dinv128 = _tc1_call(x, deg2, W1)
    acc1 = _scat_call(xs1.reshape(NC * N, HH), srcR2, dstR)
    xs2 = _tcmid_call(acc1.reshape(NC, N, HH), dinv128, A1, B1, W2)
    acc2 = _scat_call(xs2.reshape(NC * N, HH), srcR2, dstR)
    xs3 = _tcmid_call(acc2.reshape(NC, N, HH), dinv128, A2, B2, W3)
    acc3 = _scat_call(xs3.reshape(NC * N, HH), srcR2, dstR)
    return _pool_call(acc3.reshape(NC, N, HH), dinv128, b3r, batch3)


# speedup vs baseline: 1.1117x; 1.1117x over previous
import functools

import jax
import jax.numpy as jnp
from jax import lax
from jax.experimental import pallas as pl
from jax.experimental.pallas import tpu as pltpu
from jax.experimental.pallas import tpu_sc as plsc

N = 10000
E = 320000
F_IN = 128
H = 256
NG = 64
NC = 2
NS = 16
HH = H // 2

CW = 32
ROWS = E // CW
RPT = ROWS // NS
GRP = 5
NGRP = RPT // GRP
NPAIR = NGRP // 2

CWD = 40
ROWS_D = E // CWD
RPT_D = ROWS_D // (NC * NS)
NGRP_D = RPT_D // GRP
NPT = 624
TAIL = N - NS * NPT

BN_R = 1000
NBLK = N // BN_R

_MESH = dict(core_axis_name="c", subcore_axis_name="s",
             num_cores=NC, num_subcores=NS)


def _part_copy(src, dst, s, src_base, dst_base):
    pltpu.sync_copy(src.at[pl.ds(src_base + s * NPT, NPT)],
                    dst.at[pl.ds(dst_base + s * NPT, NPT)])

    @pl.when(s == NS - 1)
    def _():
        pltpu.sync_copy(src.at[pl.ds(src_base + NS * NPT, TAIL)],
                        dst.at[pl.ds(dst_base + NS * NPT, TAIL)])


def _deg_body(dstR, z16, ones80, out, acc, ones_v, didx, sem_i, sem_s):
    c = lax.axis_index("c")
    s = lax.axis_index("s")
    w = c * NS + s
    _part_copy(z16, acc, s, 0, 0)
    pltpu.sync_copy(ones80, ones_v)
    plsc.subcore_barrier()

    def grp(g, carry):
        base = w * RPT_D + g * GRP
        pltpu.async_copy(dstR.at[pl.ds(base, GRP)], didx, sem_i).wait()
        ss = [pltpu.async_copy(ones_v, acc.at[didx.at[b, 0]], sem_s, add=True)
              for b in range(GRP)]
        for d in ss:
            d.wait()
        return carry

    lax.fori_loop(0, NGRP_D, grp, 0)
    plsc.subcore_barrier()
    _part_copy(acc, out, s, 0, c * N)


def _deg_call(dstR, z16, ones80):
    return pl.kernel(
        _deg_body,
        out_type=jax.ShapeDtypeStruct((NC * N, 128), jnp.float32),
        mesh=plsc.VectorSubcoreMesh(**_MESH),
        scratch_types=[
            pltpu.VMEM_SHARED((N, 128), jnp.float32),
            pltpu.VMEM((CWD, 128), jnp.float32),
            pltpu.VMEM((GRP, 1, CWD), jnp.int32),
            pltpu.SemaphoreType.DMA,
            pltpu.SemaphoreType.DMA,
        ],
    )(dstR, z16, ones80)


def _scat_body(xs, srcR2, dstR, out, acc,
               sidxA, didxA, bufsA, sidxB, didxB, bufsB,
               sem_i, sem_g, sem_sa, sem_sb):
    c = lax.axis_index("c")
    s = lax.axis_index("s")
    row0 = s * RPT

    def idx_load(sidx, didx, base):
        d1 = pltpu.async_copy(srcR2.at[c, pl.ds(base, GRP)], sidx, sem_i)
        d2 = pltpu.async_copy(dstR.at[pl.ds(base, GRP)], didx, sem_i)
        d1.wait()
        d2.wait()

    def gathers(sidx, bufs):
        for b in range(GRP):
            pltpu.async_copy(xs.at[sidx.at[b, 0]], bufs[b], sem_g)

    def wait_gathers(sidx, bufs):
        for b in range(GRP):
            pltpu.make_async_copy(xs.at[sidx.at[b, 0]], bufs[b], sem_g).wait()

    def scatters(didx, bufs, sem):
        return [pltpu.async_copy(bufs[b], acc.at[didx.at[b, 0]], sem,
                                 add=True)
                for b in range(GRP)]

    idx_load(sidxA, didxA, row0)
    gathers(sidxA, bufsA)
    idx_load(sidxB, didxB, row0 + GRP)
    _part_copy(xs, acc, s, c * N, 0)
    plsc.subcore_barrier()

    def pair(i, carry):
        base_a2 = row0 + (2 * i + 2) * GRP
        base_a2 = lax.min(base_a2, ROWS - GRP)
        base_b2 = lax.min(base_a2 + GRP, ROWS - GRP)
        wait_gathers(sidxA, bufsA)
        sA = scatters(didxA, bufsA, sem_sa)
        gathers(sidxB, bufsB)
        for d in sA:
            d.wait()
        idx_load(sidxA, didxA, base_a2)
        wait_gathers(sidxB, bufsB)
        sB = scatters(didxB, bufsB, sem_sb)
        gathers(sidxA, bufsA)
        for d in sB:
            d.wait()
        idx_load(sidxB, didxB, base_b2)
        return carry

    lax.fori_loop(0, NPAIR, pair, 0)
    wait_gathers(sidxA, bufsA)
    for d in scatters(didxA, bufsA, sem_sa):
        d.wait()
    plsc.subcore_barrier()
    _part_copy(acc, out, s, 0, c * N)


def _scat_call(xs_flat, srcR2, dstR):
    return pl.kernel(
        _scat_body,
        out_type=jax.ShapeDtypeStruct((NC * N, HH), jnp.float32),
        mesh=plsc.VectorSubcoreMesh(**_MESH),
        scratch_types=[
            pltpu.VMEM_SHARED((N, HH), jnp.float32),
            pltpu.VMEM((GRP, 1, CW), jnp.int32),
            pltpu.VMEM((GRP, 1, CW), jnp.int32),
            [pltpu.VMEM((CW, HH), jnp.float32) for _ in range(GRP)],
            pltpu.VMEM((GRP, 1, CW), jnp.int32),
            pltpu.VMEM((GRP, 1, CW), jnp.int32),
            [pltpu.VMEM((CW, HH), jnp.float32) for _ in range(GRP)],
            pltpu.SemaphoreType.DMA,
            pltpu.SemaphoreType.DMA,
            pltpu.SemaphoreType.DMA,
            pltpu.SemaphoreType.DMA,
        ],
    )(xs_flat, srcR2, dstR)


def _tc1_body(x_ref, dega_ref, degb_ref, w1_ref, xs_ref, dinv_ref):
    deg = dega_ref[:, :1] + degb_ref[:, :1] + 1.0
    dinv = lax.rsqrt(deg)
    hw = jnp.dot(x_ref[...], w1_ref[...], preferred_element_type=jnp.float32)
    xs = hw * dinv
    xs_ref[0] = xs[:, :HH]
    xs_ref[1] = xs[:, HH:]
    dinv_ref[...] = jnp.broadcast_to(dinv, (BN_R, HH))


def _tc1_call(x, deg2, W1):
    return pl.pallas_call(
        _tc1_body,
        grid=(NBLK,),
        in_specs=[
            pl.BlockSpec((BN_R, F_IN), lambda i: (i, 0)),
            pl.BlockSpec((BN_R, 128), lambda i: (i, 0)),
            pl.BlockSpec((BN_R, 128), lambda i: (i + NBLK, 0)),
            pl.BlockSpec((F_IN, H), lambda i: (0, 0)),
        ],
        out_specs=[
            pl.BlockSpec((NC, BN_R, HH), lambda i: (0, i, 0)),
            pl.BlockSpec((BN_R, HH), lambda i: (i, 0)),
        ],
        out_shape=[
            jax.ShapeDtypeStruct((NC, N, HH), jnp.float32),
            jax.ShapeDtypeStruct((N, HH), jnp.float32),
        ],
    )(x, deg2, deg2, W1)


def _tcmid_body(acc_ref, dinv_ref, a_ref, b_ref, w_ref, xs_ref):
    dinv = dinv_ref[:, :1]
    y0 = acc_ref[0] * dinv * a_ref[:, :HH] + b_ref[:, :HH]
    y1 = acc_ref[1] * dinv * a_ref[:, HH:] + b_ref[:, HH:]
    h = jax.nn.relu(jnp.concatenate([y0, y1], axis=1))
    hw = jnp.dot(h, w_ref[...], preferred_element_type=jnp.float32)
    xs = hw * dinv
    xs_ref[0] = xs[:, :HH]
    xs_ref[1] = xs[:, HH:]


def _tcmid_call(acc3d, dinv128, A, B, W):
    return pl.pallas_call(
        _tcmid_body,
        grid=(NBLK,),
        in_specs=[
            pl.BlockSpec((NC, BN_R, HH), lambda i: (0, i, 0)),
            pl.BlockSpec((BN_R, HH), lambda i: (i, 0)),
            pl.BlockSpec((1, H), lambda i: (0, 0)),
            pl.BlockSpec((1, H), lambda i: (0, 0)),
            pl.BlockSpec((H, H), lambda i: (0, 0)),
        ],
        out_specs=pl.BlockSpec((NC, BN_R, HH), lambda i: (0, i, 0)),
        out_shape=jax.ShapeDtypeStruct((NC, N, HH), jnp.float32),
    )(acc3d, dinv128, A, B, W)


def _pool_body(acc_ref, dinv_ref, b3_ref, batch_ref, out_ref, sums, cnts):
    i = pl.program_id(0)
    dinv = dinv_ref[:, :1]
    o0 = acc_ref[0] * dinv + b3_ref[:, :HH]
    o1 = acc_ref[1] * dinv + b3_ref[:, HH:]
    out3 = jnp.concatenate([o0, o1], axis=1)
    ids = batch_ref[0, 0, :]
    gid = lax.broadcasted_iota(jnp.int32, (NG, BN_R), 0)
    oh = (gid == jnp.broadcast_to(ids[None, :], (NG, BN_R))
          ).astype(jnp.float32)

    @pl.when(i == 0)
    def _():
        sums[...] = jnp.zeros_like(sums)
        cnts[...] = jnp.zeros_like(cnts)

    sums[...] += jnp.dot(oh, out3, preferred_element_type=jnp.float32)
    cnts[...] += jnp.broadcast_to(
        jnp.sum(oh, axis=1, keepdims=True), (NG, HH))

    @pl.when(i == NBLK - 1)
    def _():
        out_ref[...] = sums[...] / jnp.maximum(cnts[:, :1], 1.0)


def _pool_call(acc3d, dinv128, b3r, batch3):
    return pl.pallas_call(
        _pool_body,
        grid=(NBLK,),
        in_specs=[
            pl.BlockSpec((NC, BN_R, HH), lambda i: (0, i, 0)),
            pl.BlockSpec((BN_R, HH), lambda i: (i, 0)),
            pl.BlockSpec((1, H), lambda i: (0, 0)),
            pl.BlockSpec((1, 1, BN_R), lambda i: (i, 0, 0)),
        ],
        out_specs=pl.BlockSpec((NG, H), lambda i: (0, 0)),
        out_shape=jax.ShapeDtypeStruct((NG, H), jnp.float32),
        scratch_shapes=[
            pltpu.VMEM((NG, H), jnp.float32),
            pltpu.VMEM((NG, HH), jnp.float32),
        ],
    )(acc3d, dinv128, b3r, batch3)


@jax.jit
def kernel(x, edge_index, batch, W1, b1, W2, b2, W3, b3,
           g1, be1, rm1, rv1, g2, be2, rm2, rv2):
    src = edge_index[0].reshape(ROWS, 1, CW)
    dstR = edge_index[1].reshape(ROWS, 1, CW)
    dstRd = edge_index[1].reshape(ROWS_D, 1, CWD)
    srcR2 = jnp.stack([src, src + N])
    batch3 = batch.reshape(NBLK, 1, BN_R)
    z16 = jnp.zeros((N, 128), jnp.float32)
    ones80 = jnp.ones((CWD, 128), jnp.float32)

    s1 = g1 * lax.rsqrt(rv1 + 1e-5)
    A1 = s1.reshape(1, H)
    B1 = ((b1 - rm1) * s1 + be1).reshape(1, H)
    s2 = g2 * lax.rsqrt(rv2 + 1e-5)
    A2 = s2.reshape(1, H)
    B2 = ((b2 - rm2) * s2 + be2).reshape(1, H)
    b3r = b3.reshape(1, H)

    deg2 = _deg_call(dstRd, z16, ones80)
    xs1, dinv128 = _tc1_call(x, deg2, W1)
    acc1 = _scat_call(xs1.reshape(NC * N, HH), srcR2, dstR)
    xs2 = _tcmid_call(acc1.reshape(NC, N, HH), dinv128, A1, B1, W2)
    acc2 = _scat_call(xs2.reshape(NC * N, HH), srcR2, dstR)
    xs3 = _tcmid_call(acc2.reshape(NC, N, HH), dinv128, A2, B2, W3)
    acc3 = _scat_call(xs3.reshape(NC * N, HH), srcR2, dstR)
    return _pool_call(acc3.reshape(NC, N, HH), dinv128, b3r, batch3)


# speedup vs baseline: 1.1215x; 1.0088x over previous
import functools

import jax
import jax.numpy as jnp
from jax import lax
from jax.experimental import pallas as pl
from jax.experimental.pallas import tpu as pltpu
from jax.experimental.pallas import tpu_sc as plsc

N = 10000
E = 320000
F_IN = 128
H = 256
NG = 64
NC = 2
NS = 16
HH = H // 2

CW = 32
ROWS = E // CW
RPT = ROWS // NS
GRP = 5
NGRP = RPT // GRP
NPAIR = NGRP // 2

CWD = 40
ROWS_D = E // CWD
RPT_D = ROWS_D // (NC * NS)
NGRP_D = RPT_D // GRP
NPT = 624
TAIL = N - NS * NPT

BN_R = 1000
NBLK = N // BN_R

_MESH = dict(core_axis_name="c", subcore_axis_name="s",
             num_cores=NC, num_subcores=NS)


def _part_copy(src, dst, s, src_base, dst_base):
    pltpu.sync_copy(src.at[pl.ds(src_base + s * NPT, NPT)],
                    dst.at[pl.ds(dst_base + s * NPT, NPT)])

    @pl.when(s == NS - 1)
    def _():
        pltpu.sync_copy(src.at[pl.ds(src_base + NS * NPT, TAIL)],
                        dst.at[pl.ds(dst_base + NS * NPT, TAIL)])


def _deg_body(dstR, z16, ones80, out, acc, ones_v, didx, sem_i, sem_s):
    c = lax.axis_index("c")
    s = lax.axis_index("s")
    w = c * NS + s
    _part_copy(z16, acc, s, 0, 0)
    pltpu.sync_copy(ones80, ones_v)
    plsc.subcore_barrier()

    def grp(g, carry):
        base = w * RPT_D + g * GRP
        pltpu.async_copy(dstR.at[pl.ds(base, GRP)], didx, sem_i).wait()
        ss = [pltpu.async_copy(ones_v, acc.at[didx.at[b, 0]], sem_s, add=True)
              for b in range(GRP)]
        for d in ss:
            d.wait()
        return carry

    lax.fori_loop(0, NGRP_D, grp, 0)
    plsc.subcore_barrier()
    _part_copy(acc, out, s, 0, c * N)


def _deg_call(dstR, z16, ones80):
    return pl.kernel(
        _deg_body,
        out_type=jax.ShapeDtypeStruct((NC * N, 16), jnp.float32),
        mesh=plsc.VectorSubcoreMesh(**_MESH),
        compiler_params=pltpu.CompilerParams(use_tc_tiling_on_sc=False),
        scratch_types=[
            pltpu.VMEM_SHARED((N, 16), jnp.float32),
            pltpu.VMEM((CWD, 16), jnp.float32),
            pltpu.VMEM((GRP, 1, CWD), jnp.int32),
            pltpu.SemaphoreType.DMA,
            pltpu.SemaphoreType.DMA,
        ],
    )(dstR, z16, ones80)


def _scat_body(xs, srcR2, dstR, out, acc,
               sidxA, didxA, bufsA, sidxB, didxB, bufsB,
               sem_i, sem_g, sem_sa, sem_sb):
    c = lax.axis_index("c")
    s = lax.axis_index("s")
    row0 = s * RPT

    def idx_load(sidx, didx, base):
        d1 = pltpu.async_copy(srcR2.at[c, pl.ds(base, GRP)], sidx, sem_i)
        d2 = pltpu.async_copy(dstR.at[pl.ds(base, GRP)], didx, sem_i)
        d1.wait()
        d2.wait()

    def gathers(sidx, bufs):
        for b in range(GRP):
            pltpu.async_copy(xs.at[sidx.at[b, 0]], bufs[b], sem_g)

    def wait_gathers(sidx, bufs):
        for b in range(GRP):
            pltpu.make_async_copy(xs.at[sidx.at[b, 0]], bufs[b], sem_g).wait()

    def scatters(didx, bufs, sem):
        return [pltpu.async_copy(bufs[b], acc.at[didx.at[b, 0]], sem,
                                 add=True)
                for b in range(GRP)]

    idx_load(sidxA, didxA, row0)
    gathers(sidxA, bufsA)
    idx_load(sidxB, didxB, row0 + GRP)
    _part_copy(xs, acc, s, c * N, 0)
    plsc.subcore_barrier()

    def pair(i, carry):
        base_a2 = row0 + (2 * i + 2) * GRP
        base_a2 = lax.min(base_a2, ROWS - GRP)
        base_b2 = lax.min(base_a2 + GRP, ROWS - GRP)
        wait_gathers(sidxA, bufsA)
        sA = scatters(didxA, bufsA, sem_sa)
        gathers(sidxB, bufsB)
        for d in sA:
            d.wait()
        idx_load(sidxA, didxA, base_a2)
        wait_gathers(sidxB, bufsB)
        sB = scatters(didxB, bufsB, sem_sb)
        gathers(sidxA, bufsA)
        for d in sB:
            d.wait()
        idx_load(sidxB, didxB, base_b2)
        return carry

    lax.fori_loop(0, NPAIR, pair, 0)
    wait_gathers(sidxA, bufsA)
    for d in scatters(didxA, bufsA, sem_sa):
        d.wait()
    plsc.subcore_barrier()
    _part_copy(acc, out, s, 0, c * N)


def _scat_call(xs_flat, srcR2, dstR):
    return pl.kernel(
        _scat_body,
        out_type=jax.ShapeDtypeStruct((NC * N, HH), jnp.float32),
        mesh=plsc.VectorSubcoreMesh(**_MESH),
        scratch_types=[
            pltpu.VMEM_SHARED((N, HH), jnp.float32),
            pltpu.VMEM((GRP, 1, CW), jnp.int32),
            pltpu.VMEM((GRP, 1, CW), jnp.int32),
            [pltpu.VMEM((CW, HH), jnp.float32) for _ in range(GRP)],
            pltpu.VMEM((GRP, 1, CW), jnp.int32),
            pltpu.VMEM((GRP, 1, CW), jnp.int32),
            [pltpu.VMEM((CW, HH), jnp.float32) for _ in range(GRP)],
            pltpu.SemaphoreType.DMA,
            pltpu.SemaphoreType.DMA,
            pltpu.SemaphoreType.DMA,
            pltpu.SemaphoreType.DMA,
        ],
    )(xs_flat, srcR2, dstR)


def _tc1_body(x_ref, dega_ref, degb_ref, w1_ref, xs_ref, dinv_ref):
    deg = dega_ref[:, :1] + degb_ref[:, :1] + 1.0
    dinv = lax.rsqrt(deg)
    hw = jnp.dot(x_ref[...], w1_ref[...], preferred_element_type=jnp.float32)
    xs = hw * dinv
    xs_ref[0] = xs[:, :HH]
    xs_ref[1] = xs[:, HH:]
    dinv_ref[...] = jnp.broadcast_to(dinv, (BN_R, HH))


def _tc1_call(x, deg2, W1):
    return pl.pallas_call(
        _tc1_body,
        grid=(NBLK,),
        in_specs=[
            pl.BlockSpec((BN_R, F_IN), lambda i: (i, 0)),
            pl.BlockSpec((BN_R, 16), lambda i: (i, 0)),
            pl.BlockSpec((BN_R, 16), lambda i: (i + NBLK, 0)),
            pl.BlockSpec((F_IN, H), lambda i: (0, 0)),
        ],
        out_specs=[
            pl.BlockSpec((NC, BN_R, HH), lambda i: (0, i, 0)),
            pl.BlockSpec((BN_R, HH), lambda i: (i, 0)),
        ],
        out_shape=[
            jax.ShapeDtypeStruct((NC, N, HH), jnp.float32),
            jax.ShapeDtypeStruct((N, HH), jnp.float32),
        ],
    )(x, deg2, deg2, W1)


def _tcmid_body(acc_ref, dinv_ref, a_ref, b_ref, w_ref, xs_ref):
    dinv = dinv_ref[:, :1]
    y0 = acc_ref[0] * dinv * a_ref[:, :HH] + b_ref[:, :HH]
    y1 = acc_ref[1] * dinv * a_ref[:, HH:] + b_ref[:, HH:]
    h = jax.nn.relu(jnp.concatenate([y0, y1], axis=1))
    hw = jnp.dot(h, w_ref[...], preferred_element_type=jnp.float32)
    xs = hw * dinv
    xs_ref[0] = xs[:, :HH]
    xs_ref[1] = xs[:, HH:]


def _tcmid_call(acc3d, dinv128, A, B, W):
    return pl.pallas_call(
        _tcmid_body,
        grid=(NBLK,),
        in_specs=[
            pl.BlockSpec((NC, BN_R, HH), lambda i: (0, i, 0)),
            pl.BlockSpec((BN_R, HH), lambda i: (i, 0)),
            pl.BlockSpec((1, H), lambda i: (0, 0)),
            pl.BlockSpec((1, H), lambda i: (0, 0)),
            pl.BlockSpec((H, H), lambda i: (0, 0)),
        ],
        out_specs=pl.BlockSpec((NC, BN_R, HH), lambda i: (0, i, 0)),
        out_shape=jax.ShapeDtypeStruct((NC, N, HH), jnp.float32),
    )(acc3d, dinv128, A, B, W)


def _pool_body(acc_ref, dinv_ref, b3_ref, batch_ref, out_ref, sums, cnts):
    i = pl.program_id(0)
    dinv = dinv_ref[:, :1]
    o0 = acc_ref[0] * dinv + b3_ref[:, :HH]
    o1 = acc_ref[1] * dinv + b3_ref[:, HH:]
    out3 = jnp.concatenate([o0, o1], axis=1)
    ids = batch_ref[0, 0, :]
    gid = lax.broadcasted_iota(jnp.int32, (NG, BN_R), 0)
    oh = (gid == jnp.broadcast_to(ids[None, :], (NG, BN_R))
          ).astype(jnp.float32)

    @pl.when(i == 0)
    def _():
        sums[...] = jnp.zeros_like(sums)
        cnts[...] = jnp.zeros_like(cnts)

    sums[...] += jnp.dot(oh, out3, preferred_element_type=jnp.float32)
    cnts[...] += jnp.broadcast_to(
        jnp.sum(oh, axis=1, keepdims=True), (NG, HH))

    @pl.when(i == NBLK - 1)
    def _():
        out_ref[...] = sums[...] / jnp.maximum(cnts[:, :1], 1.0)


def _pool_call(acc3d, dinv128, b3r, batch3):
    return pl.pallas_call(
        _pool_body,
        grid=(NBLK,),
        in_specs=[
            pl.BlockSpec((NC, BN_R, HH), lambda i: (0, i, 0)),
            pl.BlockSpec((BN_R, HH), lambda i: (i, 0)),
            pl.BlockSpec((1, H), lambda i: (0, 0)),
            pl.BlockSpec((1, 1, BN_R), lambda i: (i, 0, 0)),
        ],
        out_specs=pl.BlockSpec((NG, H), lambda i: (0, 0)),
        out_shape=jax.ShapeDtypeStruct((NG, H), jnp.float32),
        scratch_shapes=[
            pltpu.VMEM((NG, H), jnp.float32),
            pltpu.VMEM((NG, HH), jnp.float32),
        ],
    )(acc3d, dinv128, b3r, batch3)


@jax.jit
def kernel(x, edge_index, batch, W1, b1, W2, b2, W3, b3,
           g1, be1, rm1, rv1, g2, be2, rm2, rv2):
    src = edge_index[0].reshape(ROWS, 1, CW)
    dstR = edge_index[1].reshape(ROWS, 1, CW)
    dstRd = edge_index[1].reshape(ROWS_D, 1, CWD)
    srcR2 = jnp.stack([src, src + N])
    batch3 = batch.reshape(NBLK, 1, BN_R)
    z16 = jnp.zeros((N, 16), jnp.float32)
    ones80 = jnp.ones((CWD, 16), jnp.float32)

    s1 = g1 * lax.rsqrt(rv1 + 1e-5)
    A1 = s1.reshape(1, H)
    B1 = ((b1 - rm1) * s1 + be1).reshape(1, H)
    s2 = g2 * lax.rsqrt(rv2 + 1e-5)
    A2 = s2.reshape(1, H)
    B2 = ((b2 - rm2) * s2 + be2).reshape(1, H)
    b3r = b3.reshape(1, H)

    deg2 = _deg_call(dstRd, z16, ones80)
    xs1, dinv128 = _tc1_call(x, deg2, W1)
    acc1 = _scat_call(xs1.reshape(NC * N, HH), srcR2, dstR)
    xs2 = _tcmid_call(acc1.reshape(NC, N, HH), dinv128, A1, B1, W2)
    acc2 = _scat_call(xs2.reshape(NC * N, HH), srcR2, dstR)
    xs3 = _tcmid_call(acc2.reshape(NC, N, HH), dinv128, A2, B2, W3)
    acc3 = _scat_call(xs3.reshape(NC * N, HH), srcR2, dstR)
    return _pool_call(acc3.reshape(NC, N, HH), dinv128, b3r, batch3)


# speedup vs baseline: 1.2027x; 1.0724x over previous
import functools

import jax
import jax.numpy as jnp
from jax import lax
from jax.experimental import pallas as pl
from jax.experimental.pallas import tpu as pltpu
from jax.experimental.pallas import tpu_sc as plsc

N = 10000
E = 320000
F_IN = 128
H = 256
NG = 64
NC = 2
NS = 16
HH = H // 2

CW = 32
ROWS = E // CW
RPT = ROWS // NS
GRP = 5
NGRP = RPT // GRP
NPAIR = NGRP // 2

CWD = 125
ROWS_D = E // CWD
RPT_D = ROWS_D // (NC * NS)
GRP_D = 10
NGRP_D = RPT_D // GRP_D
NPT = 624
TAIL = N - NS * NPT

BN_R = 1000
NBLK = N // BN_R

_MESH = dict(core_axis_name="c", subcore_axis_name="s",
             num_cores=NC, num_subcores=NS)


def _part_copy(src, dst, s, src_base, dst_base):
    pltpu.sync_copy(src.at[pl.ds(src_base + s * NPT, NPT)],
                    dst.at[pl.ds(dst_base + s * NPT, NPT)])

    @pl.when(s == NS - 1)
    def _():
        pltpu.sync_copy(src.at[pl.ds(src_base + NS * NPT, TAIL)],
                        dst.at[pl.ds(dst_base + NS * NPT, TAIL)])


def _deg_body(dstR, z16, ones80, out, acc, ones_v, didx, sem_i, sem_s):
    c = lax.axis_index("c")
    s = lax.axis_index("s")
    w = c * NS + s
    _part_copy(z16, acc, s, 0, 0)
    pltpu.sync_copy(ones80, ones_v)
    plsc.subcore_barrier()

    def grp(g, carry):
        base = w * RPT_D + g * GRP_D
        pltpu.async_copy(dstR.at[pl.ds(base, GRP_D)], didx, sem_i).wait()
        ss = [pltpu.async_copy(ones_v, acc.at[didx.at[b, 0]], sem_s, add=True)
              for b in range(GRP_D)]
        for d in ss:
            d.wait()
        return carry

    lax.fori_loop(0, NGRP_D, grp, 0)
    plsc.subcore_barrier()
    _part_copy(acc, out, s, 0, c * N)


def _deg_call(dstR, z16, ones80):
    return pl.kernel(
        _deg_body,
        out_type=jax.ShapeDtypeStruct((NC * N, 16), jnp.float32),
        mesh=plsc.VectorSubcoreMesh(**_MESH),
        compiler_params=pltpu.CompilerParams(use_tc_tiling_on_sc=False),
        scratch_types=[
            pltpu.VMEM_SHARED((N, 16), jnp.float32),
            pltpu.VMEM((CWD, 16), jnp.float32),
            pltpu.VMEM((GRP_D, 1, CWD), jnp.int32),
            pltpu.SemaphoreType.DMA,
            pltpu.SemaphoreType.DMA,
        ],
    )(dstR, z16, ones80)


def _scat_body(xs, srcR2, dstR, out, acc,
               sidxA, didxA, bufsA, sidxB, didxB, bufsB,
               sem_i, sem_g, sem_sa, sem_sb):
    c = lax.axis_index("c")
    s = lax.axis_index("s")
    row0 = s * RPT

    def idx_load(sidx, didx, base):
        d1 = pltpu.async_copy(srcR2.at[c, pl.ds(base, GRP)], sidx, sem_i)
        d2 = pltpu.async_copy(dstR.at[pl.ds(base, GRP)], didx, sem_i)
        d1.wait()
        d2.wait()

    def gathers(sidx, bufs):
        for b in range(GRP):
            pltpu.async_copy(xs.at[sidx.at[b, 0]], bufs[b], sem_g)

    def wait_gathers(sidx, bufs):
        for b in range(GRP):
            pltpu.make_async_copy(xs.at[sidx.at[b, 0]], bufs[b], sem_g).wait()

    def scatters(didx, bufs, sem):
        return [pltpu.async_copy(bufs[b], acc.at[didx.at[b, 0]], sem,
                                 add=True)
                for b in range(GRP)]

    idx_load(sidxA, didxA, row0)
    gathers(sidxA, bufsA)
    idx_load(sidxB, didxB, row0 + GRP)
    _part_copy(xs, acc, s, c * N, 0)
    plsc.subcore_barrier()

    def pair(i, carry):
        base_a2 = row0 + (2 * i + 2) * GRP
        base_a2 = lax.min(base_a2, ROWS - GRP)
        base_b2 = lax.min(base_a2 + GRP, ROWS - GRP)
        wait_gathers(sidxA, bufsA)
        sA = scatters(didxA, bufsA, sem_sa)
        gathers(sidxB, bufsB)
        for d in sA:
            d.wait()
        idx_load(sidxA, didxA, base_a2)
        wait_gathers(sidxB, bufsB)
        sB = scatters(didxB, bufsB, sem_sb)
        gathers(sidxA, bufsA)
        for d in sB:
            d.wait()
        idx_load(sidxB, didxB, base_b2)
        return carry

    lax.fori_loop(0, NPAIR, pair, 0)
    wait_gathers(sidxA, bufsA)
    for d in scatters(didxA, bufsA, sem_sa):
        d.wait()
    plsc.subcore_barrier()
    _part_copy(acc, out, s, 0, c * N)


def _scat_call(xs_flat, srcR2, dstR):
    return pl.kernel(
        _scat_body,
        out_type=jax.ShapeDtypeStruct((NC * N, HH), jnp.float32),
        mesh=plsc.VectorSubcoreMesh(**_MESH),
        scratch_types=[
            pltpu.VMEM_SHARED((N, HH), jnp.float32),
            pltpu.VMEM((GRP, 1, CW), jnp.int32),
            pltpu.VMEM((GRP, 1, CW), jnp.int32),
            [pltpu.VMEM((CW, HH), jnp.float32) for _ in range(GRP)],
            pltpu.VMEM((GRP, 1, CW), jnp.int32),
            pltpu.VMEM((GRP, 1, CW), jnp.int32),
            [pltpu.VMEM((CW, HH), jnp.float32) for _ in range(GRP)],
            pltpu.SemaphoreType.DMA,
            pltpu.SemaphoreType.DMA,
            pltpu.SemaphoreType.DMA,
            pltpu.SemaphoreType.DMA,
        ],
    )(xs_flat, srcR2, dstR)


def _tc1_body(x_ref, dega_ref, degb_ref, w1_ref, xs_ref, dinv_ref):
    deg = dega_ref[:, :1] + degb_ref[:, :1] + 1.0
    dinv = lax.rsqrt(deg)
    hw = jnp.dot(x_ref[...], w1_ref[...], preferred_element_type=jnp.float32)
    xs = hw * dinv
    xs_ref[0] = xs[:, :HH]
    xs_ref[1] = xs[:, HH:]
    dinv_ref[...] = jnp.broadcast_to(dinv, (BN_R, HH))


def _tc1_call(x, deg2, W1):
    return pl.pallas_call(
        _tc1_body,
        grid=(NBLK,),
        in_specs=[
            pl.BlockSpec((BN_R, F_IN), lambda i: (i, 0)),
            pl.BlockSpec((BN_R, 16), lambda i: (i, 0)),
            pl.BlockSpec((BN_R, 16), lambda i: (i + NBLK, 0)),
            pl.BlockSpec((F_IN, H), lambda i: (0, 0)),
        ],
        out_specs=[
            pl.BlockSpec((NC, BN_R, HH), lambda i: (0, i, 0)),
            pl.BlockSpec((BN_R, HH), lambda i: (i, 0)),
        ],
        out_shape=[
            jax.ShapeDtypeStruct((NC, N, HH), jnp.float32),
            jax.ShapeDtypeStruct((N, HH), jnp.float32),
        ],
    )(x, deg2, deg2, W1)


def _tcmid_body(acc_ref, dinv_ref, a_ref, b_ref, w_ref, xs_ref):
    dinv = dinv_ref[:, :1]
    y0 = acc_ref[0] * dinv * a_ref[:, :HH] + b_ref[:, :HH]
    y1 = acc_ref[1] * dinv * a_ref[:, HH:] + b_ref[:, HH:]
    h = jax.nn.relu(jnp.concatenate([y0, y1], axis=1))
    hw = jnp.dot(h, w_ref[...], preferred_element_type=jnp.float32)
    xs = hw * dinv
    xs_ref[0] = xs[:, :HH]
    xs_ref[1] = xs[:, HH:]


def _tcmid_call(acc3d, dinv128, A, B, W):
    return pl.pallas_call(
        _tcmid_body,
        grid=(NBLK,),
        in_specs=[
            pl.BlockSpec((NC, BN_R, HH), lambda i: (0, i, 0)),
            pl.BlockSpec((BN_R, HH), lambda i: (i, 0)),
            pl.BlockSpec((1, H), lambda i: (0, 0)),
            pl.BlockSpec((1, H), lambda i: (0, 0)),
            pl.BlockSpec((H, H), lambda i: (0, 0)),
        ],
        out_specs=pl.BlockSpec((NC, BN_R, HH), lambda i: (0, i, 0)),
        out_shape=jax.ShapeDtypeStruct((NC, N, HH), jnp.float32),
    )(acc3d, dinv128, A, B, W)


def _pool_body(acc_ref, dinv_ref, b3_ref, batch_ref, out_ref, sums, cnts):
    i = pl.program_id(0)
    dinv = dinv_ref[:, :1]
    o0 = acc_ref[0] * dinv + b3_ref[:, :HH]
    o1 = acc_ref[1] * dinv + b3_ref[:, HH:]
    out3 = jnp.concatenate([o0, o1], axis=1)
    ids = batch_ref[0, 0, :]
    gid = lax.broadcasted_iota(jnp.int32, (NG, BN_R), 0)
    oh = (gid == jnp.broadcast_to(ids[None, :], (NG, BN_R))
          ).astype(jnp.float32)

    @pl.when(i == 0)
    def _():
        sums[...] = jnp.zeros_like(sums)
        cnts[...] = jnp.zeros_like(cnts)

    sums[...] += jnp.dot(oh, out3, preferred_element_type=jnp.float32)
    cnts[...] += jnp.broadcast_to(
        jnp.sum(oh, axis=1, keepdims=True), (NG, HH))

    @pl.when(i == NBLK - 1)
    def _():
        out_ref[...] = sums[...] / jnp.maximum(cnts[:, :1], 1.0)


def _pool_call(acc3d, dinv128, b3r, batch3):
    return pl.pallas_call(
        _pool_body,
        grid=(NBLK,),
        in_specs=[
            pl.BlockSpec((NC, BN_R, HH), lambda i: (0, i, 0)),
            pl.BlockSpec((BN_R, HH), lambda i: (i, 0)),
            pl.BlockSpec((1, H), lambda i: (0, 0)),
            pl.BlockSpec((1, 1, BN_R), lambda i: (i, 0, 0)),
        ],
        out_specs=pl.BlockSpec((NG, H), lambda i: (0, 0)),
        out_shape=jax.ShapeDtypeStruct((NG, H), jnp.float32),
        scratch_shapes=[
            pltpu.VMEM((NG, H), jnp.float32),
            pltpu.VMEM((NG, HH), jnp.float32),
        ],
    )(acc3d, dinv128, b3r, batch3)


@jax.jit
def kernel(x, edge_index, batch, W1, b1, W2, b2, W3, b3,
           g1, be1, rm1, rv1, g2, be2, rm2, rv2):
    src = edge_index[0].reshape(ROWS, 1, CW)
    dstR = edge_index[1].reshape(ROWS, 1, CW)
    dstRd = edge_index[1].reshape(ROWS_D, 1, CWD)
    srcR2 = jnp.stack([src, src + N])
    batch3 = batch.reshape(NBLK, 1, BN_R)
    z16 = jnp.zeros((N, 16), jnp.float32)
    ones80 = jnp.ones((CWD, 16), jnp.float32)

    s1 = g1 * lax.rsqrt(rv1 + 1e-5)
    A1 = s1.reshape(1, H)
    B1 = ((b1 - rm1) * s1 + be1).reshape(1, H)
    s2 = g2 * lax.rsqrt(rv2 + 1e-5)
    A2 = s2.reshape(1, H)
    B2 = ((b2 - rm2) * s2 + be2).reshape(1, H)
    b3r = b3.reshape(1, H)

    deg2 = _deg_call(dstRd, z16, ones80)
    xs1, dinv128 = _tc1_call(x, deg2, W1)
    acc1 = _scat_call(xs1.reshape(NC * N, HH), srcR2, dstR)
    xs2 = _tcmid_call(acc1.reshape(NC, N, HH), dinv128, A1, B1, W2)
    acc2 = _scat_call(xs2.reshape(NC * N, HH), srcR2, dstR)
    xs3 = _tcmid_call(acc2.reshape(NC, N, HH), dinv128, A2, B2, W3)
    acc3 = _scat_call(xs3.reshape(NC * N, HH), srcR2, dstR)
    return _pool_call(acc3.reshape(NC, N, HH), dinv128, b3r, batch3)


# speedup vs baseline: 1.2282x; 1.0212x over previous
import functools

import jax
import jax.numpy as jnp
from jax import lax
from jax.experimental import pallas as pl
from jax.experimental.pallas import tpu as pltpu
from jax.experimental.pallas import tpu_sc as plsc

N = 10000
E = 320000
F_IN = 128
H = 256
NG = 64
NC = 2
NS = 16
HH = H // 2

CW = 80
ROWS = E // CW
RPT = ROWS // NS
GRP = 2
NGRP = RPT // GRP
NPAIR = NGRP // 2

CWD = 125
ROWS_D = E // CWD
RPT_D = ROWS_D // (NC * NS)
GRP_D = 10
NGRP_D = RPT_D // GRP_D
NPT = 624
TAIL = N - NS * NPT

BN_R = 1000
NBLK = N // BN_R

_MESH = dict(core_axis_name="c", subcore_axis_name="s",
             num_cores=NC, num_subcores=NS)


def _part_copy(src, dst, s, src_base, dst_base):
    pltpu.sync_copy(src.at[pl.ds(src_base + s * NPT, NPT)],
                    dst.at[pl.ds(dst_base + s * NPT, NPT)])

    @pl.when(s == NS - 1)
    def _():
        pltpu.sync_copy(src.at[pl.ds(src_base + NS * NPT, TAIL)],
                        dst.at[pl.ds(dst_base + NS * NPT, TAIL)])


def _deg_body(dstR, z16, ones80, out, acc, ones_v, didx, sem_i, sem_s):
    c = lax.axis_index("c")
    s = lax.axis_index("s")
    w = c * NS + s
    _part_copy(z16, acc, s, 0, 0)
    pltpu.sync_copy(ones80, ones_v)
    plsc.subcore_barrier()

    def grp(g, carry):
        base = w * RPT_D + g * GRP_D
        pltpu.async_copy(dstR.at[pl.ds(base, GRP_D)], didx, sem_i).wait()
        ss = [pltpu.async_copy(ones_v, acc.at[didx.at[b, 0]], sem_s, add=True)
              for b in range(GRP_D)]
        for d in ss:
            d.wait()
        return carry

    lax.fori_loop(0, NGRP_D, grp, 0)
    plsc.subcore_barrier()
    _part_copy(acc, out, s, 0, c * N)


def _deg_call(dstR, z16, ones80):
    return pl.kernel(
        _deg_body,
        out_type=jax.ShapeDtypeStruct((NC * N, 16), jnp.float32),
        mesh=plsc.VectorSubcoreMesh(**_MESH),
        compiler_params=pltpu.CompilerParams(use_tc_tiling_on_sc=False),
        scratch_types=[
            pltpu.VMEM_SHARED((N, 16), jnp.float32),
            pltpu.VMEM((CWD, 16), jnp.float32),
            pltpu.VMEM((GRP_D, 1, CWD), jnp.int32),
            pltpu.SemaphoreType.DMA,
            pltpu.SemaphoreType.DMA,
        ],
    )(dstR, z16, ones80)


def _scat_body(xs, srcR2, dstR, out, acc,
               sidxA, didxA, bufsA, sidxB, didxB, bufsB,
               sem_i, sem_g, sem_sa, sem_sb):
    c = lax.axis_index("c")
    s = lax.axis_index("s")
    row0 = s * RPT

    def idx_load(sidx, didx, base):
        d1 = pltpu.async_copy(srcR2.at[c, pl.ds(base, GRP)], sidx, sem_i)
        d2 = pltpu.async_copy(dstR.at[pl.ds(base, GRP)], didx, sem_i)
        d1.wait()
        d2.wait()

    def gathers(sidx, bufs):
        for b in range(GRP):
            pltpu.async_copy(xs.at[sidx.at[b, 0]], bufs[b], sem_g)

    def wait_gathers(sidx, bufs):
        for b in range(GRP):
            pltpu.make_async_copy(xs.at[sidx.at[b, 0]], bufs[b], sem_g).wait()

    def scatters(didx, bufs, sem):
        return [pltpu.async_copy(bufs[b], acc.at[didx.at[b, 0]], sem,
                                 add=True)
                for b in range(GRP)]

    idx_load(sidxA, didxA, row0)
    gathers(sidxA, bufsA)
    idx_load(sidxB, didxB, row0 + GRP)
    _part_copy(xs, acc, s, c * N, 0)
    plsc.subcore_barrier()

    def pair(i, carry):
        base_a2 = row0 + (2 * i + 2) * GRP
        base_a2 = lax.min(base_a2, ROWS - GRP)
        base_b2 = lax.min(base_a2 + GRP, ROWS - GRP)
        wait_gathers(sidxA, bufsA)
        sA = scatters(didxA, bufsA, sem_sa)
        gathers(sidxB, bufsB)
        for d in sA:
            d.wait()
        idx_load(sidxA, didxA, base_a2)
        wait_gathers(sidxB, bufsB)
        sB = scatters(didxB, bufsB, sem_sb)
        gathers(sidxA, bufsA)
        for d in sB:
            d.wait()
        idx_load(sidxB, didxB, base_b2)
        return carry

    lax.fori_loop(0, NPAIR, pair, 0)
    wait_gathers(sidxA, bufsA)
    for d in scatters(didxA, bufsA, sem_sa):
        d.wait()
    plsc.subcore_barrier()
    _part_copy(acc, out, s, 0, c * N)


def _scat_call(xs_flat, srcR2, dstR):
    return pl.kernel(
        _scat_body,
        out_type=jax.ShapeDtypeStruct((NC * N, HH), jnp.float32),
        mesh=plsc.VectorSubcoreMesh(**_MESH),
        scratch_types=[
            pltpu.VMEM_SHARED((N, HH), jnp.float32),
            pltpu.VMEM((GRP, 1, CW), jnp.int32),
            pltpu.VMEM((GRP, 1, CW), jnp.int32),
            [pltpu.VMEM((CW, HH), jnp.float32) for _ in range(GRP)],
            pltpu.VMEM((GRP, 1, CW), jnp.int32),
            pltpu.VMEM((GRP, 1, CW), jnp.int32),
            [pltpu.VMEM((CW, HH), jnp.float32) for _ in range(GRP)],
            pltpu.SemaphoreType.DMA,
            pltpu.SemaphoreType.DMA,
            pltpu.SemaphoreType.DMA,
            pltpu.SemaphoreType.DMA,
        ],
    )(xs_flat, srcR2, dstR)


def _tc1_body(x_ref, dega_ref, degb_ref, w1_ref, xs_ref, dinv_ref):
    deg = dega_ref[:, :1] + degb_ref[:, :1] + 1.0
    dinv = lax.rsqrt(deg)
    hw = jnp.dot(x_ref[...], w1_ref[...], preferred_element_type=jnp.float32)
    xs = hw * dinv
    xs_ref[0] = xs[:, :HH]
    xs_ref[1] = xs[:, HH:]
    dinv_ref[...] = jnp.broadcast_to(dinv, (BN_R, HH))


def _tc1_call(x, deg2, W1):
    return pl.pallas_call(
        _tc1_body,
        grid=(NBLK,),
        in_specs=[
            pl.BlockSpec((BN_R, F_IN), lambda i: (i, 0)),
            pl.BlockSpec((BN_R, 16), lambda i: (i, 0)),
            pl.BlockSpec((BN_R, 16), lambda i: (i + NBLK, 0)),
            pl.BlockSpec((F_IN, H), lambda i: (0, 0)),
        ],
        out_specs=[
            pl.BlockSpec((NC, BN_R, HH), lambda i: (0, i, 0)),
            pl.BlockSpec((BN_R, HH), lambda i: (i, 0)),
        ],
        out_shape=[
            jax.ShapeDtypeStruct((NC, N, HH), jnp.float32),
            jax.ShapeDtypeStruct((N, HH), jnp.float32),
        ],
    )(x, deg2, deg2, W1)


def _tcmid_body(acc_ref, dinv_ref, a_ref, b_ref, w_ref, xs_ref):
    dinv = dinv_ref[:, :1]
    y0 = acc_ref[0] * dinv * a_ref[:, :HH] + b_ref[:, :HH]
    y1 = acc_ref[1] * dinv * a_ref[:, HH:] + b_ref[:, HH:]
    h = jax.nn.relu(jnp.concatenate([y0, y1], axis=1))
    hw = jnp.dot(h, w_ref[...], preferred_element_type=jnp.float32)
    xs = hw * dinv
    xs_ref[0] = xs[:, :HH]
    xs_ref[1] = xs[:, HH:]


def _tcmid_call(acc3d, dinv128, A, B, W):
    return pl.pallas_call(
        _tcmid_body,
        grid=(NBLK,),
        in_specs=[
            pl.BlockSpec((NC, BN_R, HH), lambda i: (0, i, 0)),
            pl.BlockSpec((BN_R, HH), lambda i: (i, 0)),
            pl.BlockSpec((1, H), lambda i: (0, 0)),
            pl.BlockSpec((1, H), lambda i: (0, 0)),
            pl.BlockSpec((H, H), lambda i: (0, 0)),
        ],
        out_specs=pl.BlockSpec((NC, BN_R, HH), lambda i: (0, i, 0)),
        out_shape=jax.ShapeDtypeStruct((NC, N, HH), jnp.float32),
    )(acc3d, dinv128, A, B, W)


def _pool_body(acc_ref, dinv_ref, b3_ref, batch_ref, out_ref, sums, cnts):
    i = pl.program_id(0)
    dinv = dinv_ref[:, :1]
    o0 = acc_ref[0] * dinv + b3_ref[:, :HH]
    o1 = acc_ref[1] * dinv + b3_ref[:, HH:]
    out3 = jnp.concatenate([o0, o1], axis=1)
    ids = batch_ref[0, 0, :]
    gid = lax.broadcasted_iota(jnp.int32, (NG, BN_R), 0)
    oh = (gid == jnp.broadcast_to(ids[None, :], (NG, BN_R))
          ).astype(jnp.float32)

    @pl.when(i == 0)
    def _():
        sums[...] = jnp.zeros_like(sums)
        cnts[...] = jnp.zeros_like(cnts)

    sums[...] += jnp.dot(oh, out3, preferred_element_type=jnp.float32)
    cnts[...] += jnp.broadcast_to(
        jnp.sum(oh, axis=1, keepdims=True), (NG, HH))

    @pl.when(i == NBLK - 1)
    def _():
        out_ref[...] = sums[...] / jnp.maximum(cnts[:, :1], 1.0)


def _pool_call(acc3d, dinv128, b3r, batch3):
    return pl.pallas_call(
        _pool_body,
        grid=(NBLK,),
        in_specs=[
            pl.BlockSpec((NC, BN_R, HH), lambda i: (0, i, 0)),
            pl.BlockSpec((BN_R, HH), lambda i: (i, 0)),
            pl.BlockSpec((1, H), lambda i: (0, 0)),
            pl.BlockSpec((1, 1, BN_R), lambda i: (i, 0, 0)),
        ],
        out_specs=pl.BlockSpec((NG, H), lambda i: (0, 0)),
        out_shape=jax.ShapeDtypeStruct((NG, H), jnp.float32),
        scratch_shapes=[
            pltpu.VMEM((NG, H), jnp.float32),
            pltpu.VMEM((NG, HH), jnp.float32),
        ],
    )(acc3d, dinv128, b3r, batch3)


@jax.jit
def kernel(x, edge_index, batch, W1, b1, W2, b2, W3, b3,
           g1, be1, rm1, rv1, g2, be2, rm2, rv2):
    src = edge_index[0].reshape(ROWS, 1, CW)
    dstR = edge_index[1].reshape(ROWS, 1, CW)
    dstRd = edge_index[1].reshape(ROWS_D, 1, CWD)
    srcR2 = jnp.stack([src, src + N])
    batch3 = batch.reshape(NBLK, 1, BN_R)
    z16 = jnp.zeros((N, 16), jnp.float32)
    ones80 = jnp.ones((CWD, 16), jnp.float32)

    s1 = g1 * lax.rsqrt(rv1 + 1e-5)
    A1 = s1.reshape(1, H)
    B1 = ((b1 - rm1) * s1 + be1).reshape(1, H)
    s2 = g2 * lax.rsqrt(rv2 + 1e-5)
    A2 = s2.reshape(1, H)
    B2 = ((b2 - rm2) * s2 + be2).reshape(1, H)
    b3r = b3.reshape(1, H)

    deg2 = _deg_call(dstRd, z16, ones80)
    xs1, dinv128 = _tc1_call(x, deg2, W1)
    acc1 = _scat_call(xs1.reshape(NC * N, HH), srcR2, dstR)
    xs2 = _tcmid_call(acc1.reshape(NC, N, HH), dinv128, A1, B1, W2)
    acc2 = _scat_call(xs2.reshape(NC * N, HH), srcR2, dstR)
    xs3 = _tcmid_call(acc2.reshape(NC, N, HH), dinv128, A2, B2, W3)
    acc3 = _scat_call(xs3.reshape(NC * N, HH), srcR2, dstR)
    return _pool_call(acc3.reshape(NC, N, HH), dinv128, b3r, batch3)


# speedup vs baseline: 1.2291x; 1.0007x over previous
import functools

import jax
import jax.numpy as jnp
from jax import lax
from jax.experimental import pallas as pl
from jax.experimental.pallas import tpu as pltpu
from jax.experimental.pallas import tpu_sc as plsc

N = 10000
E = 320000
F_IN = 128
H = 256
NG = 64
NC = 2
NS = 16
HH = H // 2

CW = 80
ROWS = E // CW
RPT = ROWS // NS
GRP = 2
NGRP = RPT // GRP
NPAIR = NGRP // 2

CWD = 125
ROWS_D = E // CWD
RPT_D = ROWS_D // (NC * NS)
GRP_D = 20
NGRP_D = RPT_D // GRP_D
NPT = 624
TAIL = N - NS * NPT

BN_R = 1000
NBLK = N // BN_R

_MESH = dict(core_axis_name="c", subcore_axis_name="s",
             num_cores=NC, num_subcores=NS)


def _part_copy(src, dst, s, src_base, dst_base):
    pltpu.sync_copy(src.at[pl.ds(src_base + s * NPT, NPT)],
                    dst.at[pl.ds(dst_base + s * NPT, NPT)])

    @pl.when(s == NS - 1)
    def _():
        pltpu.sync_copy(src.at[pl.ds(src_base + NS * NPT, TAIL)],
                        dst.at[pl.ds(dst_base + NS * NPT, TAIL)])


def _deg_body(dstR, z16, ones80, out, acc, ones_v, didx, sem_i, sem_s):
    c = lax.axis_index("c")
    s = lax.axis_index("s")
    w = c * NS + s
    _part_copy(z16, acc, s, 0, 0)
    pltpu.sync_copy(ones80, ones_v)
    plsc.subcore_barrier()

    def grp(g, carry):
        base = w * RPT_D + g * GRP_D
        pltpu.async_copy(dstR.at[pl.ds(base, GRP_D)], didx, sem_i).wait()
        ss = [pltpu.async_copy(ones_v, acc.at[didx.at[b, 0]], sem_s, add=True)
              for b in range(GRP_D)]
        for d in ss:
            d.wait()
        return carry

    lax.fori_loop(0, NGRP_D, grp, 0)
    plsc.subcore_barrier()
    _part_copy(acc, out, s, 0, c * N)


def _deg_call(dstR, z16, ones80):
    return pl.kernel(
        _deg_body,
        out_type=jax.ShapeDtypeStruct((NC * N, 16), jnp.float32),
        mesh=plsc.VectorSubcoreMesh(**_MESH),
        compiler_params=pltpu.CompilerParams(use_tc_tiling_on_sc=False),
        scratch_types=[
            pltpu.VMEM_SHARED((N, 16), jnp.float32),
            pltpu.VMEM((CWD, 16), jnp.float32),
            pltpu.VMEM((GRP_D, 1, CWD), jnp.int32),
            pltpu.SemaphoreType.DMA,
            pltpu.SemaphoreType.DMA,
        ],
    )(dstR, z16, ones80)


def _scat_body(xs, srcR2, dstR, out, acc,
               sidxA, didxA, bufsA, sidxB, didxB, bufsB,
               sem_i, sem_g, sem_sa, sem_sb):
    c = lax.axis_index("c")
    s = lax.axis_index("s")
    row0 = s * RPT

    def idx_load(sidx, didx, base):
        d1 = pltpu.async_copy(srcR2.at[c, pl.ds(base, GRP)], sidx, sem_i)
        d2 = pltpu.async_copy(dstR.at[pl.ds(base, GRP)], didx, sem_i)
        d1.wait()
        d2.wait()

    def gathers(sidx, bufs):
        for b in range(GRP):
            pltpu.async_copy(xs.at[sidx.at[b, 0]], bufs[b], sem_g)

    def wait_gathers(sidx, bufs):
        for b in range(GRP):
            pltpu.make_async_copy(xs.at[sidx.at[b, 0]], bufs[b], sem_g).wait()

    def scatters(didx, bufs, sem):
        return [pltpu.async_copy(bufs[b], acc.at[didx.at[b, 0]], sem,
                                 add=True)
                for b in range(GRP)]

    idx_load(sidxA, didxA, row0)
    gathers(sidxA, bufsA)
    idx_load(sidxB, didxB, row0 + GRP)
    _part_copy(xs, acc, s, c * N, 0)
    plsc.subcore_barrier()

    def pair(i, carry):
        base_a2 = row0 + (2 * i + 2) * GRP
        base_a2 = lax.min(base_a2, ROWS - GRP)
        base_b2 = lax.min(base_a2 + GRP, ROWS - GRP)
        wait_gathers(sidxA, bufsA)
        sA = scatters(didxA, bufsA, sem_sa)
        gathers(sidxB, bufsB)
        for d in sA:
            d.wait()
        idx_load(sidxA, didxA, base_a2)
        wait_gathers(sidxB, bufsB)
        sB = scatters(didxB, bufsB, sem_sb)
        gathers(sidxA, bufsA)
        for d in sB:
            d.wait()
        idx_load(sidxB, didxB, base_b2)
        return carry

    lax.fori_loop(0, NPAIR, pair, 0)
    wait_gathers(sidxA, bufsA)
    for d in scatters(didxA, bufsA, sem_sa):
        d.wait()
    plsc.subcore_barrier()
    _part_copy(acc, out, s, 0, c * N)


def _scat_call(xs_flat, srcR2, dstR):
    return pl.kernel(
        _scat_body,
        out_type=jax.ShapeDtypeStruct((NC * N, HH), jnp.float32),
        mesh=plsc.VectorSubcoreMesh(**_MESH),
        scratch_types=[
            pltpu.VMEM_SHARED((N, HH), jnp.float32),
            pltpu.VMEM((GRP, 1, CW), jnp.int32),
            pltpu.VMEM((GRP, 1, CW), jnp.int32),
            [pltpu.VMEM((CW, HH), jnp.float32) for _ in range(GRP)],
            pltpu.VMEM((GRP, 1, CW), jnp.int32),
            pltpu.VMEM((GRP, 1, CW), jnp.int32),
            [pltpu.VMEM((CW, HH), jnp.float32) for _ in range(GRP)],
            pltpu.SemaphoreType.DMA,
            pltpu.SemaphoreType.DMA,
            pltpu.SemaphoreType.DMA,
            pltpu.SemaphoreType.DMA,
        ],
    )(xs_flat, srcR2, dstR)


def _tc1_body(x_ref, dega_ref, degb_ref, w1_ref, xs_ref, dinv_ref):
    deg = dega_ref[:, :1] + degb_ref[:, :1] + 1.0
    dinv = lax.rsqrt(deg)
    hw = jnp.dot(x_ref[...], w1_ref[...], preferred_element_type=jnp.float32)
    xs = hw * dinv
    xs_ref[0] = xs[:, :HH]
    xs_ref[1] = xs[:, HH:]
    dinv_ref[...] = jnp.broadcast_to(dinv, (BN_R, HH))


def _tc1_call(x, deg2, W1):
    return pl.pallas_call(
        _tc1_body,
        grid=(NBLK,),
        in_specs=[
            pl.BlockSpec((BN_R, F_IN), lambda i: (i, 0)),
            pl.BlockSpec((BN_R, 16), lambda i: (i, 0)),
            pl.BlockSpec((BN_R, 16), lambda i: (i + NBLK, 0)),
            pl.BlockSpec((F_IN, H), lambda i: (0, 0)),
        ],
        out_specs=[
            pl.BlockSpec((NC, BN_R, HH), lambda i: (0, i, 0)),
            pl.BlockSpec((BN_R, HH), lambda i: (i, 0)),
        ],
        out_shape=[
            jax.ShapeDtypeStruct((NC, N, HH), jnp.float32),
            jax.ShapeDtypeStruct((N, HH), jnp.float32),
        ],
    )(x, deg2, deg2, W1)


def _tcmid_body(acc_ref, dinv_ref, a_ref, b_ref, w_ref, xs_ref):
    dinv = dinv_ref[:, :1]
    y0 = acc_ref[0] * dinv * a_ref[:, :HH] + b_ref[:, :HH]
    y1 = acc_ref[1] * dinv * a_ref[:, HH:] + b_ref[:, HH:]
    h = jax.nn.relu(jnp.concatenate([y0, y1], axis=1))
    hw = jnp.dot(h, w_ref[...], preferred_element_type=jnp.float32)
    xs = hw * dinv
    xs_ref[0] = xs[:, :HH]
    xs_ref[1] = xs[:, HH:]


def _tcmid_call(acc3d, dinv128, A, B, W):
    return pl.pallas_call(
        _tcmid_body,
        grid=(NBLK,),
        in_specs=[
            pl.BlockSpec((NC, BN_R, HH), lambda i: (0, i, 0)),
            pl.BlockSpec((BN_R, HH), lambda i: (i, 0)),
            pl.BlockSpec((1, H), lambda i: (0, 0)),
            pl.BlockSpec((1, H), lambda i: (0, 0)),
            pl.BlockSpec((H, H), lambda i: (0, 0)),
        ],
        out_specs=pl.BlockSpec((NC, BN_R, HH), lambda i: (0, i, 0)),
        out_shape=jax.ShapeDtypeStruct((NC, N, HH), jnp.float32),
    )(acc3d, dinv128, A, B, W)


def _pool_body(acc_ref, dinv_ref, b3_ref, batch_ref, out_ref, sums, cnts):
    i = pl.program_id(0)
    dinv = dinv_ref[:, :1]
    o0 = acc_ref[0] * dinv + b3_ref[:, :HH]
    o1 = acc_ref[1] * dinv + b3_ref[:, HH:]
    out3 = jnp.concatenate([o0, o1], axis=1)
    ids = batch_ref[0, 0, :]
    gid = lax.broadcasted_iota(jnp.int32, (NG, BN_R), 0)
    oh = (gid == jnp.broadcast_to(ids[None, :], (NG, BN_R))
          ).astype(jnp.float32)

    @pl.when(i == 0)
    def _():
        sums[...] = jnp.zeros_like(sums)
        cnts[...] = jnp.zeros_like(cnts)

    sums[...] += jnp.dot(oh, out3, preferred_element_type=jnp.float32)
    cnts[...] += jnp.broadcast_to(
        jnp.sum(oh, axis=1, keepdims=True), (NG, HH))

    @pl.when(i == NBLK - 1)
    def _():
        out_ref[...] = sums[...] / jnp.maximum(cnts[:, :1], 1.0)


def _pool_call(acc3d, dinv128, b3r, batch3):
    return pl.pallas_call(
        _pool_body,
        grid=(NBLK,),
        in_specs=[
            pl.BlockSpec((NC, BN_R, HH), lambda i: (0, i, 0)),
            pl.BlockSpec((BN_R, HH), lambda i: (i, 0)),
            pl.BlockSpec((1, H), lambda i: (0, 0)),
            pl.BlockSpec((1, 1, BN_R), lambda i: (i, 0, 0)),
        ],
        out_specs=pl.BlockSpec((NG, H), lambda i: (0, 0)),
        out_shape=jax.ShapeDtypeStruct((NG, H), jnp.float32),
        scratch_shapes=[
            pltpu.VMEM((NG, H), jnp.float32),
            pltpu.VMEM((NG, HH), jnp.float32),
        ],
    )(acc3d, dinv128, b3r, batch3)


@jax.jit
def kernel(x, edge_index, batch, W1, b1, W2, b2, W3, b3,
           g1, be1, rm1, rv1, g2, be2, rm2, rv2):
    src = edge_index[0].reshape(ROWS, 1, CW)
    dstR = edge_index[1].reshape(ROWS, 1, CW)
    dstRd = edge_index[1].reshape(ROWS_D, 1, CWD)
    srcR2 = jnp.stack([src, src + N])
    batch3 = batch.reshape(NBLK, 1, BN_R)
    z16 = jnp.zeros((N, 16), jnp.float32)
    ones80 = jnp.ones((CWD, 16), jnp.float32)

    s1 = g1 * lax.rsqrt(rv1 + 1e-5)
    A1 = s1.reshape(1, H)
    B1 = ((b1 - rm1) * s1 + be1).reshape(1, H)
    s2 = g2 * lax.rsqrt(rv2 + 1e-5)
    A2 = s2.reshape(1, H)
    B2 = ((b2 - rm2) * s2 + be2).reshape(1, H)
    b3r = b3.reshape(1, H)

    deg2 = _deg_call(dstRd, z16, ones80)
    xs1, dinv128 = _tc1_call(x, deg2, W1)
    acc1 = _scat_call(xs1.reshape(NC * N, HH), srcR2, dstR)
    xs2 = _tcmid_call(acc1.reshape(NC, N, HH), dinv128, A1, B1, W2)
    acc2 = _scat_call(xs2.reshape(NC * N, HH), srcR2, dstR)
    xs3 = _tcmid_call(acc2.reshape(NC, N, HH), dinv128, A2, B2, W3)
    acc3 = _scat_call(xs3.reshape(NC * N, HH), srcR2, dstR)
    return _pool_call(acc3.reshape(NC, N, HH), dinv128, b3r, batch3)
